# baseline (speedup 1.0000x reference)
import jax
import jax.numpy as jnp
from jax.experimental import pallas as pl
from jax.experimental.pallas import tpu as pltpu

K_IN = 900
OUT_W = 98


def _probe(x_ref, o_ref):
    o_ref[...] = x_ref[:, :OUT_W]


@jax.jit
def kernel(x, wmat, gamma, beta):
    n = x.shape[0]
    tile_n = 1024
    num_tiles = n // tile_n
    x_flat = x.reshape(n, K_IN)
    return pl.pallas_call(
        _probe,
        out_shape=jax.ShapeDtypeStruct((n, OUT_W), jnp.float32),
        grid=(num_tiles,),
        in_specs=[pl.BlockSpec((tile_n, K_IN), lambda i: (i, 0))],
        out_specs=pl.BlockSpec((tile_n, OUT_W), lambda i: (i, 0)),
        compiler_params=pltpu.CompilerParams(
            dimension_semantics=("arbitrary",),
            vmem_limit_bytes=60 * 1024 * 1024,
        ),
    )(x_flat)


# speedup vs baseline: 1.2781x; 1.2781x over previous
import functools

import jax
import jax.numpy as jnp
from jax.experimental import pallas as pl
from jax.experimental.pallas import tpu as pltpu

K_IN = 900
OUT_W = 98
PAD_W = 128
BN_EPS = 1e-5
SP_THR = 20.0


def _fused_t(x_ref, w_ref, g_ref, b_ref, o_ref,
             conv_buf, s1, s2, scale, shift, *, n, num_tiles):
    p = pl.program_id(0)
    i = pl.program_id(1)

    @pl.when((p == 0) & (i == 0))
    def _init():
        s1[...] = jnp.zeros_like(s1)
        s2[...] = jnp.zeros_like(s2)

    @pl.when(p == 0)
    def _conv_stats():
        c = jax.lax.dot_general(
            w_ref[...], x_ref[...],
            dimension_numbers=(((0,), (0,)), ((), ())),
            preferred_element_type=jnp.float32)
        conv_buf[i] = c
        s1[...] += c
        s2[...] += c * c

    @pl.when((p == 0) & (i == num_tiles - 1))
    def _finalize():
        inv_n = jnp.float32(1.0 / n)
        mean = jnp.sum(s1[...], axis=1, keepdims=True) * inv_n
        ex2 = jnp.sum(s2[...], axis=1, keepdims=True) * inv_n
        var = jnp.maximum(ex2 - mean * mean, 0.0)
        sc = g_ref[...] * jax.lax.rsqrt(var + BN_EPS)
        scale[...] = sc
        shift[...] = b_ref[...] - mean * sc

    @pl.when(p == 1)
    def _bn_softplus():
        y = conv_buf[i] * scale[...] + shift[...]
        sp = jnp.log1p(jnp.exp(jnp.minimum(y, SP_THR)))
        o_ref[...] = jnp.where(y > SP_THR, y, sp)[:OUT_W, :]


@jax.jit
def kernel(x, wmat, gamma, beta):
    n = x.shape[0]
    tile_l = 2048 if n % 2048 == 0 else 128
    num_tiles = n // tile_l

    xt = x.reshape(n, K_IN).T

    g_c = jnp.zeros((PAD_W, 1), jnp.float32).at[:OUT_W, 0].set(
        gamma.astype(jnp.float32).reshape(-1))
    b_c = jnp.zeros((PAD_W, 1), jnp.float32).at[:OUT_W, 0].set(
        beta.astype(jnp.float32).reshape(-1))

    out_t = pl.pallas_call(
        functools.partial(_fused_t, n=n, num_tiles=num_tiles),
        out_shape=jax.ShapeDtypeStruct((OUT_W, n), jnp.float32),
        grid=(2, num_tiles),
        in_specs=[
            pl.BlockSpec((K_IN, tile_l),
                         lambda p, i: (0, i * (1 - p) + (num_tiles - 1) * p)),
            pl.BlockSpec((K_IN, PAD_W), lambda p, i: (0, 0)),
            pl.BlockSpec((PAD_W, 1), lambda p, i: (0, 0)),
            pl.BlockSpec((PAD_W, 1), lambda p, i: (0, 0)),
        ],
        out_specs=pl.BlockSpec((OUT_W, tile_l), lambda p, i: (0, i * p)),
        scratch_shapes=[
            pltpu.VMEM((num_tiles, PAD_W, tile_l), jnp.float32),
            pltpu.VMEM((PAD_W, tile_l), jnp.float32),
            pltpu.VMEM((PAD_W, tile_l), jnp.float32),
            pltpu.VMEM((PAD_W, 1), jnp.float32),
            pltpu.VMEM((PAD_W, 1), jnp.float32),
        ],
        compiler_params=pltpu.CompilerParams(
            dimension_semantics=("arbitrary", "arbitrary"),
            vmem_limit_bytes=60 * 1024 * 1024,
        ),
    )(xt, wmat, g_c, b_c)

    return out_t.T
